# baseline (speedup 1.0000x reference)
import jax
import jax.numpy as jnp
from jax import lax
from jax.experimental import pallas as pl
from jax.experimental.pallas import tpu as pltpu
from jax.experimental.pallas import tpu_sc as plsc

VOCAB = 100000
EMB = 128
COORD = 4
B = 16384

NC = 2
NS = 16
NW = NC * NS
BPW = B // NW
NK = BPW // 128
L = 16


def _sc_kernel(table_hbm, idx_hbm, co_hbm, wb_hbm, rows_hbm, ce_hbm,
               idx_v, rows_v, co_v, ce_v, wb_v, gsem):
    wid = lax.axis_index("s") * NC + lax.axis_index("c")
    base = wid * BPW

    pltpu.sync_copy(idx_hbm.at[pl.ds(wid * NK, NK), :], idx_v)
    gcopies = [
        pltpu.async_copy(table_hbm.at[idx_v.at[k]],
                         rows_v.at[pl.ds(k * 128, 128), :], gsem)
        for k in range(NK)
    ]

    pltpu.sync_copy(wb_hbm, wb_v)
    pltpu.sync_copy(co_hbm.at[:, pl.ds(base, BPW)], co_v)

    wrow = wb_v[pl.ds(0, L)]
    brow = wb_v[pl.ds(L, L)]
    wsp = [[wrow[4 * i + j] for j in range(COORD)] for i in range(COORD)]
    bsp = [brow[i] for i in range(COORD)]

    @pl.loop(0, BPW, step=L)
    def _(c0):
        cols = [co_v[j, pl.ds(c0, L)] for j in range(COORD)]
        for i in range(COORD):
            z = cols[0] * wsp[i][0] + bsp[i]
            for j in range(1, COORD):
                z = z + cols[j] * wsp[i][j]
            e = jnp.exp(z + z)
            t = 1.0 - 2.0 / (e + 1.0)
            ce_v[i, pl.ds(c0, L)] = cols[i] + t

    pltpu.sync_copy(ce_v, ce_hbm.at[:, pl.ds(base, BPW)])

    for gcopy in gcopies:
        gcopy.wait()
    pltpu.sync_copy(rows_v, rows_hbm.at[pl.ds(base, BPW), :])


@jax.jit
def kernel(x, coordinates, table, W, b):
    table_bf = jnp.pad(table, ((0, 0), (0, COORD))).astype(jnp.bfloat16)
    idx = x.reshape(B // 128, 128).astype(jnp.int32)
    wb = jnp.concatenate([W.reshape(-1), b, jnp.zeros((12,), jnp.float32)])

    mesh = plsc.VectorSubcoreMesh(core_axis_name="c", subcore_axis_name="s")
    run = pl.kernel(
        _sc_kernel,
        out_type=(jax.ShapeDtypeStruct((B, EMB), jnp.bfloat16),
                  jax.ShapeDtypeStruct((COORD, B), jnp.float32)),
        mesh=mesh,
        compiler_params=pltpu.CompilerParams(use_tc_tiling_on_sc=False,
                                             needs_layout_passes=False),
        scratch_types=[
            pltpu.VMEM((NK, 128), jnp.int32),
            pltpu.VMEM((BPW, EMB), jnp.bfloat16),
            pltpu.VMEM((COORD, BPW), jnp.float32),
            pltpu.VMEM((COORD, BPW), jnp.float32),
            pltpu.VMEM((2 * L,), jnp.float32),
            pltpu.SemaphoreType.DMA,
        ],
    )
    rows, ce = run(table_bf, idx, coordinates.T, wb)
    emb = rows[:, : EMB - COORD].astype(jnp.float32)
    out = jnp.concatenate([emb, ce.T], axis=1)
    return out.reshape(B, 1, EMB)


# speedup vs baseline: 3.2458x; 3.2458x over previous
import jax
import jax.numpy as jnp
from jax import lax
from jax.experimental import pallas as pl
from jax.experimental.pallas import tpu as pltpu
from jax.experimental.pallas import tpu_sc as plsc

VOCAB = 100000
EMB = 128
COORD = 4
B = 16384

NC = 2
NS = 16
NW = NC * NS
BPW = B // NW
NK = BPW // 128
L = 16


def _sc_kernel(table_hbm, idx_hbm, co_hbm, wb_hbm, out_hbm,
               idx_v, rows_v, co_v, ce_v, wb_v, gsem):
    wid = lax.axis_index("s") * NC + lax.axis_index("c")
    base = wid * BPW

    pltpu.sync_copy(idx_hbm.at[pl.ds(wid * NK, NK), :], idx_v)
    gcopies = [
        pltpu.async_copy(table_hbm.at[idx_v.at[k]],
                         rows_v.at[pl.ds(k * 128, 128), :], gsem)
        for k in range(NK)
    ]

    pltpu.sync_copy(wb_hbm, wb_v)
    pltpu.sync_copy(co_hbm.at[:, pl.ds(base, BPW)], co_v)

    wrow = wb_v[pl.ds(0, L)]
    brow = wb_v[pl.ds(L, L)]
    wsp = [[wrow[4 * i + j] for j in range(COORD)] for i in range(COORD)]
    bsp = [brow[i] for i in range(COORD)]

    @pl.loop(0, BPW, step=L)
    def _(c0):
        cols = [co_v[j, pl.ds(c0, L)] for j in range(COORD)]
        for i in range(COORD):
            z = cols[0] * wsp[i][0] + bsp[i]
            for j in range(1, COORD):
                z = z + cols[j] * wsp[i][j]
            e = jnp.exp(z + z)
            t = 1.0 - 2.0 / (e + 1.0)
            ce_v[i, pl.ds(c0, L)] = cols[i] + t

    for gcopy in gcopies:
        gcopy.wait()

    @pl.loop(0, BPW, step=L)
    def _(c0):
        ids = lax.iota(jnp.int32, L) + c0
        for i in range(COORD):
            plsc.store_scatter(rows_v,
                               [ids, jnp.full((L,), EMB - COORD + i, jnp.int32)],
                               ce_v[i, pl.ds(c0, L)])

    pltpu.sync_copy(rows_v, out_hbm.at[pl.ds(base, BPW), :])


@jax.jit
def kernel(x, coordinates, table, W, b):
    table128 = jnp.pad(table, ((0, 0), (0, COORD)))
    idx = x.reshape(B // 128, 128).astype(jnp.int32)
    wb = jnp.concatenate([W.reshape(-1), b, jnp.zeros((12,), jnp.float32)])

    mesh = plsc.VectorSubcoreMesh(core_axis_name="c", subcore_axis_name="s")
    run = pl.kernel(
        _sc_kernel,
        out_type=jax.ShapeDtypeStruct((B, EMB), jnp.float32),
        mesh=mesh,
        compiler_params=pltpu.CompilerParams(use_tc_tiling_on_sc=False,
                                             needs_layout_passes=False),
        scratch_types=[
            pltpu.VMEM((NK, 128), jnp.int32),
            pltpu.VMEM((BPW, EMB), jnp.float32),
            pltpu.VMEM((COORD, BPW), jnp.float32),
            pltpu.VMEM((COORD, BPW), jnp.float32),
            pltpu.VMEM((2 * L,), jnp.float32),
            pltpu.SemaphoreType.DMA,
        ],
    )
    out = run(table128, idx, coordinates.T, wb)
    return out.reshape(B, 1, EMB)
